# baseline (speedup 1.0000x reference)
import functools

import jax
import jax.numpy as jnp
from jax import lax
from jax.experimental import pallas as pl
from jax.experimental.pallas import tpu as pltpu
from jax.experimental.pallas import tpu_sc as plsc

_NC = 2
_NS = 16
_L = 16
_DIM = 64
_NCARD_IDS = 52
_NRANK = 13
_NSUIT = 4
_K = 5
_DBLK = 32


@functools.lru_cache(maxsize=None)
def _make_sc_kernel(n_rows):
    nw = _NC * _NS
    rows_per_w = n_rows // nw
    groups = rows_per_w // _L
    assert rows_per_w * nw == n_rows and groups * _L == rows_per_w

    mesh = plsc.VectorSubcoreMesh(core_axis_name="c", subcore_axis_name="s")

    @functools.partial(
        pl.kernel,
        mesh=mesh,
        out_type=jax.ShapeDtypeStruct((n_rows, _DIM), jnp.float32),
        compiler_params=pltpu.CompilerParams(needs_layout_passes=False),
        scratch_types=[
            pltpu.VMEM((_NCARD_IDS, _DIM), jnp.float32),
            pltpu.VMEM((_NRANK, _DIM), jnp.float32),
            pltpu.VMEM((_NSUIT, _DIM), jnp.float32),
        ] + [
            pltpu.VMEM((rows_per_w,), jnp.int32)
            for _ in range(_K)
        ] + [
            pltpu.VMEM((rows_per_w, _DIM), jnp.float32),
        ],
    )
    def sc_kernel(idx_hbm, card_hbm, rank_hbm, suit_hbm, out_hbm,
                  t_v, rank_v, suit_v, *idx_and_out):
        idx_vs = idx_and_out[:_K]
        out_v = idx_and_out[_K]
        wid = lax.axis_index("s") * _NC + lax.axis_index("c")
        row0 = wid * rows_per_w

        pltpu.sync_copy(card_hbm, t_v)
        pltpu.sync_copy(rank_hbm, rank_v)
        pltpu.sync_copy(suit_hbm, suit_v)
        for j in range(_K):
            pltpu.sync_copy(idx_hbm.at[pl.ds(j * n_rows + row0, rows_per_w)],
                            idx_vs[j])

        for i in range(_NCARD_IDS):
            r, s = i // 4, i % 4
            for c in range(0, _DIM, _L):
                t_v[i, pl.ds(c, _L)] = (t_v[i, pl.ds(c, _L)]
                                        + rank_v[r, pl.ds(c, _L)]
                                        + suit_v[s, pl.ds(c, _L)])

        lanes = lax.iota(jnp.int32, _L)

        @plsc.parallel_loop(0, groups, unroll=2)
        def group_body(g):
            n0 = g * _L
            rowvec = n0 + lanes
            ids = [idx_vs[j][pl.ds(n0, _L)] for j in range(_K)]
            for d0 in range(0, _DIM, _DBLK):
                accs = []
                for d in range(d0, d0 + _DBLK):
                    cvec = (lanes + d) & (_DIM - 1)
                    acc = plsc.load_gather(t_v, [ids[0], cvec])
                    for j in range(1, _K):
                        acc = acc + plsc.load_gather(t_v, [ids[j], cvec])
                    accs.append((cvec, acc))
                for cvec, acc in accs:
                    plsc.store_scatter(out_v, [rowvec, cvec], acc)

        pltpu.sync_copy(out_v, out_hbm.at[pl.ds(row0, rows_per_w)])

    return sc_kernel


def kernel(input, card, rank, suit):
    n, _ = input.shape
    idx = input.astype(jnp.int32).T.reshape(-1)
    return _make_sc_kernel(n)(idx, card, rank, suit)


# speedup vs baseline: 1.1878x; 1.1878x over previous
import functools

import jax
import jax.numpy as jnp
from jax import lax
from jax.experimental import pallas as pl
from jax.experimental.pallas import tpu as pltpu
from jax.experimental.pallas import tpu_sc as plsc

_NC = 2
_NS = 16
_L = 16
_DIM = 64
_NCARD_IDS = 52
_NRANK = 13
_NSUIT = 4
_K = 5
_DBLK = 16


@functools.lru_cache(maxsize=None)
def _make_sc_kernel(n_rows):
    nw = _NC * _NS
    rows_per_w = n_rows // nw
    groups = rows_per_w // _L
    assert rows_per_w * nw == n_rows and groups * _L == rows_per_w

    mesh = plsc.VectorSubcoreMesh(core_axis_name="c", subcore_axis_name="s")

    @functools.partial(
        pl.kernel,
        mesh=mesh,
        out_type=jax.ShapeDtypeStruct((n_rows, _DIM), jnp.float32),
        compiler_params=pltpu.CompilerParams(needs_layout_passes=False),
        scratch_types=[
            pltpu.VMEM((_NCARD_IDS, _DIM), jnp.float32),
            pltpu.VMEM((_NRANK, _DIM), jnp.float32),
            pltpu.VMEM((_NSUIT, _DIM), jnp.float32),
        ] + [
            pltpu.VMEM((rows_per_w,), jnp.int32)
            for _ in range(_K)
        ] + [
            pltpu.VMEM((rows_per_w, _DIM), jnp.float32),
        ],
    )
    def sc_kernel(idx_hbm, card_hbm, rank_hbm, suit_hbm, out_hbm,
                  t_v, rank_v, suit_v, *idx_and_out):
        idx_vs = idx_and_out[:_K]
        out_v = idx_and_out[_K]
        wid = lax.axis_index("s") * _NC + lax.axis_index("c")
        row0 = wid * rows_per_w

        pltpu.sync_copy(card_hbm, t_v)
        pltpu.sync_copy(rank_hbm, rank_v)
        pltpu.sync_copy(suit_hbm, suit_v)
        for j in range(_K):
            pltpu.sync_copy(idx_hbm.at[pl.ds(j * n_rows + row0, rows_per_w)],
                            idx_vs[j])

        for i in range(_NCARD_IDS):
            r, s = i // 4, i % 4
            for c in range(0, _DIM, _L):
                t_v[i, pl.ds(c, _L)] = (t_v[i, pl.ds(c, _L)]
                                        + rank_v[r, pl.ds(c, _L)]
                                        + suit_v[s, pl.ds(c, _L)])

        lanes = lax.iota(jnp.int32, _L)

        @plsc.parallel_loop(0, groups)
        def group_body(g):
            n0 = g * _L
            rowvec = n0 + lanes
            ids = [idx_vs[j][pl.ds(n0, _L)] for j in range(_K)]
            for d0 in range(0, _DIM, _DBLK):
                accs = []
                for d in range(d0, d0 + _DBLK):
                    cvec = (lanes + d) & (_DIM - 1)
                    acc = plsc.load_gather(t_v, [ids[0], cvec])
                    for j in range(1, _K):
                        acc = acc + plsc.load_gather(t_v, [ids[j], cvec])
                    accs.append((cvec, acc))
                for cvec, acc in accs:
                    plsc.store_scatter(out_v, [rowvec, cvec], acc)

        pltpu.sync_copy(out_v, out_hbm.at[pl.ds(row0, rows_per_w)])

    return sc_kernel


def kernel(input, card, rank, suit):
    n, _ = input.shape
    idx = input.astype(jnp.int32).T.reshape(-1)
    return _make_sc_kernel(n)(idx, card, rank, suit)


# speedup vs baseline: 1.1907x; 1.0024x over previous
import functools

import jax
import jax.numpy as jnp
from jax import lax
from jax.experimental import pallas as pl
from jax.experimental.pallas import tpu as pltpu
from jax.experimental.pallas import tpu_sc as plsc

_NC = 2
_NS = 16
_L = 16
_DIM = 64
_NCARD_IDS = 52
_NRANK = 13
_NSUIT = 4
_K = 5
_DBLK = 16


@functools.lru_cache(maxsize=None)
def _make_sc_kernel(n_rows):
    nw = _NC * _NS
    rows_per_w = n_rows // nw
    groups = rows_per_w // _L
    assert rows_per_w * nw == n_rows and groups * _L == rows_per_w

    mesh = plsc.VectorSubcoreMesh(core_axis_name="c", subcore_axis_name="s")

    @functools.partial(
        pl.kernel,
        mesh=mesh,
        out_type=jax.ShapeDtypeStruct((n_rows, _DIM), jnp.float32),
        compiler_params=pltpu.CompilerParams(needs_layout_passes=False,
                                             use_tc_tiling_on_sc=True),
        scratch_types=[
            pltpu.VMEM((_NCARD_IDS, _DIM), jnp.float32),
            pltpu.VMEM((_NRANK, _DIM), jnp.float32),
            pltpu.VMEM((_NSUIT, _DIM), jnp.float32),
        ] + [
            pltpu.VMEM((rows_per_w,), jnp.int32)
            for _ in range(_K)
        ] + [
            pltpu.VMEM((rows_per_w, _DIM), jnp.float32),
        ],
    )
    def sc_kernel(idx_hbm, card_hbm, rank_hbm, suit_hbm, out_hbm,
                  t_v, rank_v, suit_v, *idx_and_out):
        idx_vs = idx_and_out[:_K]
        out_v = idx_and_out[_K]
        wid = lax.axis_index("s") * _NC + lax.axis_index("c")
        row0 = wid * rows_per_w

        pltpu.sync_copy(card_hbm, t_v)
        pltpu.sync_copy(rank_hbm, rank_v)
        pltpu.sync_copy(suit_hbm, suit_v)
        for j in range(_K):
            pltpu.sync_copy(idx_hbm.at[pl.ds(j * n_rows + row0, rows_per_w)],
                            idx_vs[j])

        for i in range(_NCARD_IDS):
            r, s = i // 4, i % 4
            for c in range(0, _DIM, _L):
                t_v[i, pl.ds(c, _L)] = (t_v[i, pl.ds(c, _L)]
                                        + rank_v[r, pl.ds(c, _L)]
                                        + suit_v[s, pl.ds(c, _L)])

        lanes = lax.iota(jnp.int32, _L)

        @plsc.parallel_loop(0, groups)
        def group_body(g):
            n0 = g * _L
            rowvec = n0 + lanes
            ids = [idx_vs[j][pl.ds(n0, _L)] for j in range(_K)]
            for d0 in range(0, _DIM, _DBLK):
                accs = []
                for d in range(d0, d0 + _DBLK):
                    cvec = (lanes + d) & (_DIM - 1)
                    acc = plsc.load_gather(t_v, [ids[0], cvec])
                    for j in range(1, _K):
                        acc = acc + plsc.load_gather(t_v, [ids[j], cvec])
                    accs.append((cvec, acc))
                for cvec, acc in accs:
                    plsc.store_scatter(out_v, [rowvec, cvec], acc)

        pltpu.sync_copy(out_v, out_hbm.at[pl.ds(row0, rows_per_w)])

    return sc_kernel


def kernel(input, card, rank, suit):
    n, _ = input.shape
    idx = input.astype(jnp.int32).T.reshape(-1)
    return _make_sc_kernel(n)(idx, card, rank, suit)
